# baseline (speedup 1.0000x reference)
import functools

import jax
import jax.numpy as jnp
from jax import lax
from jax.experimental import pallas as pl
from jax.experimental.pallas import tpu as pltpu
from jax.experimental.pallas import tpu_sc as plsc

N = 10000
E = 320000
D = 128

NC = 2
NS = 16
NW = NC * NS
EW = E // NW
CH = 64
NCHUNK = EW // CH
TAIL = EW - NCHUNK * CH
NSET = 3
NPAD = 10112
ROWS_PER_TILE = NPAD // NS
DEN_W = 10240
DEN_WT = DEN_W // NS

_BLK = 1000
_GRID = N // _BLK


def _dense_first_body(x_ref, wt_ref, as_ref, ad_ref, h_ref, s_ref, d_ref):
    h = jnp.dot(x_ref[...], wt_ref[...], preferred_element_type=jnp.float32)
    h_ref[...] = h
    s_ref[...] = jnp.dot(h, as_ref[...], preferred_element_type=jnp.float32)
    d_ref[...] = jnp.dot(h, ad_ref[...], preferred_element_type=jnp.float32)


def _dense_mid_body(a0_ref, a1_ref, dn0_ref, dn1_ref, s_ref, d_ref, c_ref,
                    hp_ref, b_ref, wt_ref, as_ref, ad_ref,
                    h_ref, s_out_ref, d_out_ref):
    t = s_ref[...] + d_ref[...]
    t = jnp.where(t > 0, t, 0.2 * t)
    ee = jnp.exp(t - c_ref[0])
    den = dn0_ref[...] + dn1_ref[...] + ee
    acc = a0_ref[...] + a1_ref[...] + ee * hp_ref[...]
    x = jnp.maximum(acc / (den + 1e-16) + b_ref[...], 0.0)
    h = jnp.dot(x, wt_ref[...], preferred_element_type=jnp.float32)
    h_ref[...] = h
    s_out_ref[...] = jnp.dot(h, as_ref[...], preferred_element_type=jnp.float32)
    d_out_ref[...] = jnp.dot(h, ad_ref[...], preferred_element_type=jnp.float32)


def _dense_last_body(a0_ref, a1_ref, dn0_ref, dn1_ref, s_ref, d_ref, c_ref,
                     hp_ref, b_ref, out_ref):
    t = s_ref[...] + d_ref[...]
    t = jnp.where(t > 0, t, 0.2 * t)
    ee = jnp.exp(t - c_ref[0])
    den = dn0_ref[...] + dn1_ref[...] + ee
    acc = a0_ref[...] + a1_ref[...] + ee * hp_ref[...]
    out_ref[...] = jnp.tanh(acc / (den + 1e-16) + b_ref[...])


def _col_spec():
    return pl.BlockSpec((_BLK, 1), lambda i: (i, 0))


def _row_spec():
    return pl.BlockSpec((_BLK, D), lambda i: (i, 0))


def _full_spec(shape):
    return pl.BlockSpec(shape, lambda i: tuple(0 for _ in shape))


def _smem_spec():
    return pl.BlockSpec(memory_space=pltpu.SMEM)


def _dense_first(x, wt, as_col, ad_col):
    return pl.pallas_call(
        _dense_first_body,
        grid=(_GRID,),
        in_specs=[_row_spec(), _full_spec((D, D)), _full_spec((D, 1)),
                  _full_spec((D, 1))],
        out_specs=[_row_spec(), _col_spec(), _col_spec()],
        out_shape=[jax.ShapeDtypeStruct((N, D), jnp.float32),
                   jax.ShapeDtypeStruct((N, 1), jnp.float32),
                   jax.ShapeDtypeStruct((N, 1), jnp.float32)],
    )(x, wt, as_col, ad_col)


def _dense_mid(a0, a1, dn0, dn1, s, d, c, hp, b, wt, as_col, ad_col):
    return pl.pallas_call(
        _dense_mid_body,
        grid=(_GRID,),
        in_specs=[_row_spec(), _row_spec(), _col_spec(), _col_spec(),
                  _col_spec(), _col_spec(), _smem_spec(), _row_spec(),
                  _full_spec((1, D)), _full_spec((D, D)), _full_spec((D, 1)),
                  _full_spec((D, 1))],
        out_specs=[_row_spec(), _col_spec(), _col_spec()],
        out_shape=[jax.ShapeDtypeStruct((N, D), jnp.float32),
                   jax.ShapeDtypeStruct((N, 1), jnp.float32),
                   jax.ShapeDtypeStruct((N, 1), jnp.float32)],
    )(a0, a1, dn0, dn1, s, d, c, hp, b, wt, as_col, ad_col)


def _dense_last(a0, a1, dn0, dn1, s, d, c, hp, b):
    return pl.pallas_call(
        _dense_last_body,
        grid=(_GRID,),
        in_specs=[_row_spec(), _row_spec(), _col_spec(), _col_spec(),
                  _col_spec(), _col_spec(), _smem_spec(), _row_spec(),
                  _full_spec((1, D))],
        out_specs=pl.BlockSpec((_BLK, D), lambda i: (i, 0)),
        out_shape=jax.ShapeDtypeStruct((N, D), jnp.float32),
    )(a0, a1, dn0, dn1, s, d, c, hp, b)


def _compute_chunk(s_buf, d_buf, cvec, srcb, dstb, eeb, rowsb, n_edges):
    for k in range(n_edges // 16):
        srcv = srcb[pl.ds(k * 16, 16)]
        dstv = dstb[pl.ds(k * 16, 16)]
        sv = plsc.load_gather(s_buf, [srcv])
        dv = plsc.load_gather(d_buf, [dstv])
        e = sv + dv
        e = jnp.where(e > 0, e, 0.2 * e)
        ee = jnp.exp(e - cvec)
        eeb[pl.ds(k * 16, 16)] = ee

    @plsc.parallel_loop(0, n_edges, 1, unroll=16)
    def sbody(j):
        eej = plsc.load_gather(eeb, [lax.broadcast(j, (16,))])
        for cb in range(D // 16):
            sl = pl.ds(cb * 16, 16)
            rowsb[j, sl] = rowsb[j, sl] * eej


def _edge_body(h_hbm, s_hbm, d_hbm, c_hbm, src_hbm, dst_hbm, z2d_hbm, z1d_hbm,
               acc_out, den_out, s_buf, d_buf, c_buf, bufs, src_t, dst_t,
               sems, acc_sp, den_sp):
    cid = lax.axis_index("c")
    sid = lax.axis_index("s")
    wid = sid * NC + cid

    pltpu.sync_copy(s_hbm, s_buf)
    pltpu.sync_copy(d_hbm, d_buf)
    pltpu.sync_copy(c_hbm, c_buf)
    pltpu.sync_copy(z2d_hbm.at[pl.ds(sid * ROWS_PER_TILE, ROWS_PER_TILE)],
                    acc_sp.at[pl.ds(sid * ROWS_PER_TILE, ROWS_PER_TILE)])
    pltpu.sync_copy(z1d_hbm, den_sp.at[pl.ds(sid * DEN_WT, DEN_WT)])
    plsc.subcore_barrier()

    cvec = c_buf[...]
    ebase = wid * EW

    def issue_idx(t, base):
        src_b, dst_b, _, _ = bufs[t]
        is_s, is_d, _, _, _ = sems[t]
        a = pltpu.async_copy(src_hbm.at[pl.ds(base, CH)], src_b, is_s)
        b = pltpu.async_copy(dst_hbm.at[pl.ds(base, CH)], dst_b, is_d)
        return a, b

    def drain_scatters(t):
        src_b, dst_b, ee_b, rows_b = bufs[t]
        _, _, _, sc, dd = sems[t]
        pltpu.make_async_copy(rows_b, acc_sp.at[dst_b], sc).wait()
        pltpu.make_async_copy(ee_b, den_sp.at[dst_b], dd).wait()

    def issue_gather(t, ia):
        src_b, _, _, rows_b = bufs[t]
        _, _, gs, _, _ = sems[t]
        ia.wait()
        return pltpu.async_copy(h_hbm.at[src_b], rows_b, gs)

    def run_compute(t, ib, gr):
        _, dst_b, ee_b, rows_b = bufs[t]
        _, _, _, sc, dd = sems[t]
        ib.wait()
        gr.wait()
        _compute_chunk(s_buf, d_buf, cvec, bufs[t][0], dst_b, ee_b, rows_b, CH)
        a = pltpu.async_copy(rows_b, acc_sp.at[dst_b], sc, add=True)
        b = pltpu.async_copy(ee_b, den_sp.at[dst_b], dd, add=True)
        return a, b

    def triple(t3, carry):
        @pl.when(t3 > 0)
        def _():
            for t in range(NSET):
                drain_scatters(t)

        base = pl.multiple_of(ebase + (3 * t3) * CH, 8)
        ia0, ib0 = issue_idx(0, base)
        ia1, ib1 = issue_idx(1, base + CH)
        ia2, ib2 = issue_idx(2, base + 2 * CH)
        gr0 = issue_gather(0, ia0)
        gr1 = issue_gather(1, ia1)
        s0 = run_compute(0, ib0, gr0)
        gr2 = issue_gather(2, ia2)
        s1 = run_compute(1, ib1, gr1)
        s2 = run_compute(2, ib2, gr2)
        del s0, s1, s2
        return carry

    lax.fori_loop(0, NCHUNK // 3, triple, 0)
    for t in range(NSET):
        drain_scatters(t)

    if TAIL:
        _, _, ee_b, rows_b = bufs[0]
        _, _, gs, _, _ = sems[0]
        base = pl.multiple_of(ebase + NCHUNK * CH, 8)
        pltpu.sync_copy(src_hbm.at[pl.ds(base, TAIL)], src_t)
        pltpu.sync_copy(dst_hbm.at[pl.ds(base, TAIL)], dst_t)
        pltpu.async_copy(h_hbm.at[src_t], rows_b.at[pl.ds(0, TAIL)],
                         gs).wait()
        _compute_chunk(s_buf, d_buf, cvec, src_t, dst_t, ee_b, rows_b, TAIL)
        pltpu.sync_copy(rows_b.at[pl.ds(0, TAIL)], acc_sp.at[dst_t], add=True)
        pltpu.sync_copy(ee_b.at[pl.ds(0, TAIL)], den_sp.at[dst_t], add=True)
    plsc.subcore_barrier()

    r0 = sid * ROWS_PER_TILE
    pltpu.sync_copy(acc_sp.at[pl.ds(r0, ROWS_PER_TILE)],
                    acc_out.at[cid, pl.ds(r0, ROWS_PER_TILE)])
    pltpu.sync_copy(den_sp.at[pl.ds(sid * DEN_WT, DEN_WT)],
                    den_out.at[cid, pl.ds(sid * DEN_WT, DEN_WT)])


def _set_scratch():
    out = []
    for _ in range(NSET):
        out += [pltpu.VMEM((CH,), jnp.int32),
                pltpu.VMEM((CH,), jnp.int32),
                pltpu.VMEM((CH,), jnp.float32),
                pltpu.VMEM((CH, D), jnp.float32)]
    out += [pltpu.VMEM((TAIL,), jnp.int32),
            pltpu.VMEM((TAIL,), jnp.int32)]
    out += [pltpu.SemaphoreType.DMA] * (5 * NSET)
    return out


@functools.partial(
    pl.kernel,
    out_type=[jax.ShapeDtypeStruct((NC, NPAD, D), jnp.float32),
              jax.ShapeDtypeStruct((NC, DEN_W), jnp.float32)],
    mesh=plsc.VectorSubcoreMesh(core_axis_name="c", subcore_axis_name="s",
                                num_cores=NC, num_subcores=NS),
    compiler_params=pltpu.CompilerParams(needs_layout_passes=False),
    scratch_types=[
        pltpu.VMEM((N,), jnp.float32),
        pltpu.VMEM((N,), jnp.float32),
        pltpu.VMEM((16,), jnp.float32),
        pltpu.VMEM_SHARED((NPAD, D), jnp.float32),
        pltpu.VMEM_SHARED((DEN_W,), jnp.float32),
    ] + _set_scratch(),
)
def _edge_kernel(h, s, d, c, src, dst, z2d, z1d, acc_out, den_out,
                 s_buf, d_buf, c_buf, acc_sp, den_sp, *rest):
    bufs = [rest[4 * t:4 * t + 4] for t in range(NSET)]
    src_t, dst_t = rest[4 * NSET:4 * NSET + 2]
    off = 4 * NSET + 2
    sems = [rest[off + 5 * t:off + 5 * t + 5] for t in range(NSET)]
    _edge_body(h, s, d, c, src, dst, z2d, z1d, acc_out, den_out,
               s_buf, d_buf, c_buf, bufs, src_t, dst_t, sems, acc_sp, den_sp)


def kernel(x, edge_index, W0, as0, ad0, b0, W1, as1, ad1, b1, W2, as2, ad2, b2):
    src = edge_index[0]
    dst = edge_index[1]
    zeros2d = jnp.zeros((NPAD, D), jnp.float32)
    zeros1d = jnp.zeros((DEN_WT,), jnp.float32)

    layers = ((W0, as0, ad0, b0), (W1, as1, ad1, b1), (W2, as2, ad2, b2))

    h, s, d = _dense_first(x, W0.T, as0.reshape(D, 1), ad0.reshape(D, 1))

    for li in range(3):
        _, _, _, b = layers[li]
        s1 = s.reshape(N)
        d1 = d.reshape(N)
        cm = jnp.max(s1) + jnp.max(d1)
        cm = jnp.where(cm > 0, cm, 0.2 * cm)
        c16 = jnp.full((16,), cm, jnp.float32)
        acc2, den2 = _edge_kernel(h, s1, d1, c16, src, dst, zeros2d, zeros1d)
        acc2 = acc2[:, :N]
        den2 = den2[:, :N].reshape(NC, N, 1)
        c11 = cm.reshape(1)
        if li < 2:
            Wn, asn, adn, _ = layers[li + 1]
            h, s, d = _dense_mid(
                acc2[0], acc2[1], den2[0], den2[1], s.reshape(N, 1),
                d.reshape(N, 1), c11, h, b.reshape(1, D), Wn.T,
                asn.reshape(D, 1), adn.reshape(D, 1))
        else:
            out = _dense_last(
                acc2[0], acc2[1], den2[0], den2[1], s.reshape(N, 1),
                d.reshape(N, 1), c11, h, b.reshape(1, D))
    return out


# speedup vs baseline: 1.0085x; 1.0085x over previous
import functools

import jax
import jax.numpy as jnp
from jax import lax
from jax.experimental import pallas as pl
from jax.experimental.pallas import tpu as pltpu
from jax.experimental.pallas import tpu_sc as plsc

N = 10000
E = 320000
D = 128

NC = 2
NS = 16
NW = NC * NS
EW = E // NW
CH = 96
NCHUNK = EW // CH
TAIL = EW - NCHUNK * CH
NSET = 2
NPAD = 10112
ROWS_PER_TILE = NPAD // NS
DEN_W = 10240
DEN_WT = DEN_W // NS

_BLK = 1000
_GRID = N // _BLK


def _dense_first_body(x_ref, wt_ref, as_ref, ad_ref, h_ref, s_ref, d_ref):
    h = jnp.dot(x_ref[...], wt_ref[...], preferred_element_type=jnp.float32)
    h_ref[...] = h
    s_ref[...] = jnp.dot(h, as_ref[...], preferred_element_type=jnp.float32)
    d_ref[...] = jnp.dot(h, ad_ref[...], preferred_element_type=jnp.float32)


def _dense_mid_body(a0_ref, a1_ref, dn0_ref, dn1_ref, s_ref, d_ref, c_ref,
                    hp_ref, b_ref, wt_ref, as_ref, ad_ref,
                    h_ref, s_out_ref, d_out_ref):
    t = s_ref[...] + d_ref[...]
    t = jnp.where(t > 0, t, 0.2 * t)
    ee = jnp.exp(t - c_ref[0])
    den = dn0_ref[...] + dn1_ref[...] + ee
    acc = a0_ref[...] + a1_ref[...] + ee * hp_ref[...]
    x = jnp.maximum(acc / (den + 1e-16) + b_ref[...], 0.0)
    h = jnp.dot(x, wt_ref[...], preferred_element_type=jnp.float32)
    h_ref[...] = h
    s_out_ref[...] = jnp.dot(h, as_ref[...], preferred_element_type=jnp.float32)
    d_out_ref[...] = jnp.dot(h, ad_ref[...], preferred_element_type=jnp.float32)


def _dense_last_body(a0_ref, a1_ref, dn0_ref, dn1_ref, s_ref, d_ref, c_ref,
                     hp_ref, b_ref, out_ref):
    t = s_ref[...] + d_ref[...]
    t = jnp.where(t > 0, t, 0.2 * t)
    ee = jnp.exp(t - c_ref[0])
    den = dn0_ref[...] + dn1_ref[...] + ee
    acc = a0_ref[...] + a1_ref[...] + ee * hp_ref[...]
    out_ref[...] = jnp.tanh(acc / (den + 1e-16) + b_ref[...])


def _col_spec():
    return pl.BlockSpec((_BLK, 1), lambda i: (i, 0))


def _row_spec():
    return pl.BlockSpec((_BLK, D), lambda i: (i, 0))


def _full_spec(shape):
    return pl.BlockSpec(shape, lambda i: tuple(0 for _ in shape))


def _smem_spec():
    return pl.BlockSpec(memory_space=pltpu.SMEM)


def _dense_first(x, wt, as_col, ad_col):
    return pl.pallas_call(
        _dense_first_body,
        grid=(_GRID,),
        in_specs=[_row_spec(), _full_spec((D, D)), _full_spec((D, 1)),
                  _full_spec((D, 1))],
        out_specs=[_row_spec(), _col_spec(), _col_spec()],
        out_shape=[jax.ShapeDtypeStruct((N, D), jnp.float32),
                   jax.ShapeDtypeStruct((N, 1), jnp.float32),
                   jax.ShapeDtypeStruct((N, 1), jnp.float32)],
    )(x, wt, as_col, ad_col)


def _dense_mid(a0, a1, dn0, dn1, s, d, c, hp, b, wt, as_col, ad_col):
    return pl.pallas_call(
        _dense_mid_body,
        grid=(_GRID,),
        in_specs=[_row_spec(), _row_spec(), _col_spec(), _col_spec(),
                  _col_spec(), _col_spec(), _smem_spec(), _row_spec(),
                  _full_spec((1, D)), _full_spec((D, D)), _full_spec((D, 1)),
                  _full_spec((D, 1))],
        out_specs=[_row_spec(), _col_spec(), _col_spec()],
        out_shape=[jax.ShapeDtypeStruct((N, D), jnp.float32),
                   jax.ShapeDtypeStruct((N, 1), jnp.float32),
                   jax.ShapeDtypeStruct((N, 1), jnp.float32)],
    )(a0, a1, dn0, dn1, s, d, c, hp, b, wt, as_col, ad_col)


def _dense_last(a0, a1, dn0, dn1, s, d, c, hp, b):
    return pl.pallas_call(
        _dense_last_body,
        grid=(_GRID,),
        in_specs=[_row_spec(), _row_spec(), _col_spec(), _col_spec(),
                  _col_spec(), _col_spec(), _smem_spec(), _row_spec(),
                  _full_spec((1, D))],
        out_specs=pl.BlockSpec((_BLK, D), lambda i: (i, 0)),
        out_shape=jax.ShapeDtypeStruct((N, D), jnp.float32),
    )(a0, a1, dn0, dn1, s, d, c, hp, b)


def _compute_chunk(s_buf, d_buf, cvec, srcb, dstb, eeb, rowsb, n_edges):
    for k in range(n_edges // 16):
        srcv = srcb[pl.ds(k * 16, 16)]
        dstv = dstb[pl.ds(k * 16, 16)]
        sv = plsc.load_gather(s_buf, [srcv])
        dv = plsc.load_gather(d_buf, [dstv])
        e = sv + dv
        e = jnp.where(e > 0, e, 0.2 * e)
        ee = jnp.exp(e - cvec)
        eeb[pl.ds(k * 16, 16)] = ee

    @plsc.parallel_loop(0, n_edges, 1, unroll=8)
    def sbody(j):
        eej = plsc.load_gather(eeb, [lax.broadcast(j, (16,))])
        for cb in range(D // 16):
            sl = pl.ds(cb * 16, 16)
            rowsb[j, sl] = rowsb[j, sl] * eej


def _edge_body(h_hbm, s_hbm, d_hbm, c_hbm, src_hbm, dst_hbm, z2d_hbm, z1d_hbm,
               acc_out, den_out, s_buf, d_buf, c_buf, bufs, src_t, dst_t,
               sems, acc_sp, den_sp):
    cid = lax.axis_index("c")
    sid = lax.axis_index("s")
    wid = sid * NC + cid

    pltpu.sync_copy(s_hbm, s_buf)
    pltpu.sync_copy(d_hbm, d_buf)
    pltpu.sync_copy(c_hbm, c_buf)
    pltpu.sync_copy(z2d_hbm.at[pl.ds(sid * ROWS_PER_TILE, ROWS_PER_TILE)],
                    acc_sp.at[pl.ds(sid * ROWS_PER_TILE, ROWS_PER_TILE)])
    pltpu.sync_copy(z1d_hbm, den_sp.at[pl.ds(sid * DEN_WT, DEN_WT)])
    plsc.subcore_barrier()

    cvec = c_buf[...]
    ebase = wid * EW

    def issue_idx(t, base):
        src_b, dst_b, _, _ = bufs[t]
        is_s, is_d, _, _, _ = sems[t]
        a = pltpu.async_copy(src_hbm.at[pl.ds(base, CH)], src_b, is_s)
        b = pltpu.async_copy(dst_hbm.at[pl.ds(base, CH)], dst_b, is_d)
        return a, b

    def drain_scatters(t):
        src_b, dst_b, ee_b, rows_b = bufs[t]
        _, _, _, sc, dd = sems[t]
        pltpu.make_async_copy(rows_b, acc_sp.at[dst_b], sc).wait()
        pltpu.make_async_copy(ee_b, den_sp.at[dst_b], dd).wait()

    def issue_gather(t, ia):
        src_b, _, _, rows_b = bufs[t]
        _, _, gs, _, _ = sems[t]
        ia.wait()
        return pltpu.async_copy(h_hbm.at[src_b], rows_b, gs)

    def run_compute(t, ib, gr):
        _, dst_b, ee_b, rows_b = bufs[t]
        _, _, _, sc, dd = sems[t]
        ib.wait()
        gr.wait()
        _compute_chunk(s_buf, d_buf, cvec, bufs[t][0], dst_b, ee_b, rows_b, CH)
        a = pltpu.async_copy(rows_b, acc_sp.at[dst_b], sc, add=True)
        b = pltpu.async_copy(ee_b, den_sp.at[dst_b], dd, add=True)
        return a, b

    def group(tg, carry):
        @pl.when(tg > 0)
        def _():
            for t in range(NSET):
                drain_scatters(t)

        base = pl.multiple_of(ebase + (NSET * tg) * CH, 8)
        idx = [issue_idx(t, base + t * CH) for t in range(NSET)]
        grs = [issue_gather(t, idx[t][0]) for t in range(NSET)]
        for t in range(NSET):
            run_compute(t, idx[t][1], grs[t])
        return carry

    lax.fori_loop(0, NCHUNK // NSET, group, 0)
    for t in range(NSET):
        drain_scatters(t)

    if TAIL:
        _, _, ee_b, rows_b = bufs[0]
        _, _, gs, _, _ = sems[0]
        base = pl.multiple_of(ebase + NCHUNK * CH, 8)
        pltpu.sync_copy(src_hbm.at[pl.ds(base, TAIL)], src_t)
        pltpu.sync_copy(dst_hbm.at[pl.ds(base, TAIL)], dst_t)
        pltpu.async_copy(h_hbm.at[src_t], rows_b.at[pl.ds(0, TAIL)],
                         gs).wait()
        _compute_chunk(s_buf, d_buf, cvec, src_t, dst_t, ee_b, rows_b, TAIL)
        pltpu.sync_copy(rows_b.at[pl.ds(0, TAIL)], acc_sp.at[dst_t], add=True)
        pltpu.sync_copy(ee_b.at[pl.ds(0, TAIL)], den_sp.at[dst_t], add=True)
    plsc.subcore_barrier()

    r0 = sid * ROWS_PER_TILE
    pltpu.sync_copy(acc_sp.at[pl.ds(r0, ROWS_PER_TILE)],
                    acc_out.at[cid, pl.ds(r0, ROWS_PER_TILE)])
    pltpu.sync_copy(den_sp.at[pl.ds(sid * DEN_WT, DEN_WT)],
                    den_out.at[cid, pl.ds(sid * DEN_WT, DEN_WT)])


def _set_scratch():
    out = []
    for _ in range(NSET):
        out += [pltpu.VMEM((CH,), jnp.int32),
                pltpu.VMEM((CH,), jnp.int32),
                pltpu.VMEM((CH,), jnp.float32),
                pltpu.VMEM((CH, D), jnp.float32)]
    out += [pltpu.VMEM((TAIL,), jnp.int32),
            pltpu.VMEM((TAIL,), jnp.int32)]
    out += [pltpu.SemaphoreType.DMA] * (5 * NSET)
    return out


@functools.partial(
    pl.kernel,
    out_type=[jax.ShapeDtypeStruct((NC, NPAD, D), jnp.float32),
              jax.ShapeDtypeStruct((NC, DEN_W), jnp.float32)],
    mesh=plsc.VectorSubcoreMesh(core_axis_name="c", subcore_axis_name="s",
                                num_cores=NC, num_subcores=NS),
    compiler_params=pltpu.CompilerParams(needs_layout_passes=False),
    scratch_types=[
        pltpu.VMEM((N,), jnp.float32),
        pltpu.VMEM((N,), jnp.float32),
        pltpu.VMEM((16,), jnp.float32),
        pltpu.VMEM_SHARED((NPAD, D), jnp.float32),
        pltpu.VMEM_SHARED((DEN_W,), jnp.float32),
    ] + _set_scratch(),
)
def _edge_kernel(h, s, d, c, src, dst, z2d, z1d, acc_out, den_out,
                 s_buf, d_buf, c_buf, acc_sp, den_sp, *rest):
    bufs = [rest[4 * t:4 * t + 4] for t in range(NSET)]
    src_t, dst_t = rest[4 * NSET:4 * NSET + 2]
    off = 4 * NSET + 2
    sems = [rest[off + 5 * t:off + 5 * t + 5] for t in range(NSET)]
    _edge_body(h, s, d, c, src, dst, z2d, z1d, acc_out, den_out,
               s_buf, d_buf, c_buf, bufs, src_t, dst_t, sems, acc_sp, den_sp)


def kernel(x, edge_index, W0, as0, ad0, b0, W1, as1, ad1, b1, W2, as2, ad2, b2):
    src = edge_index[0]
    dst = edge_index[1]
    zeros2d = jnp.zeros((NPAD, D), jnp.float32)
    zeros1d = jnp.zeros((DEN_WT,), jnp.float32)

    layers = ((W0, as0, ad0, b0), (W1, as1, ad1, b1), (W2, as2, ad2, b2))

    h, s, d = _dense_first(x, W0.T, as0.reshape(D, 1), ad0.reshape(D, 1))

    for li in range(3):
        _, _, _, b = layers[li]
        s1 = s.reshape(N)
        d1 = d.reshape(N)
        cm = jnp.max(s1) + jnp.max(d1)
        cm = jnp.where(cm > 0, cm, 0.2 * cm)
        c16 = jnp.full((16,), cm, jnp.float32)
        acc2, den2 = _edge_kernel(h, s1, d1, c16, src, dst, zeros2d, zeros1d)
        acc2 = acc2[:, :N]
        den2 = den2[:, :N].reshape(NC, N, 1)
        c11 = cm.reshape(1)
        if li < 2:
            Wn, asn, adn, _ = layers[li + 1]
            h, s, d = _dense_mid(
                acc2[0], acc2[1], den2[0], den2[1], s.reshape(N, 1),
                d.reshape(N, 1), c11, h, b.reshape(1, D), Wn.T,
                asn.reshape(D, 1), adn.reshape(D, 1))
        else:
            out = _dense_last(
                acc2[0], acc2[1], den2[0], den2[1], s.reshape(N, 1),
                d.reshape(N, 1), c11, h, b.reshape(1, D))
    return out
